# baseline (speedup 1.0000x reference)
import functools
import jax
import jax.numpy as jnp
from jax import lax
from jax.experimental import pallas as pl
from jax.experimental.pallas import tpu as pltpu
from jax.experimental.pallas import tpu_sc as plsc

N = 10000
E = 320000
F = 128
C = 64

NC = 2
NS = 16
NW = NC * NS

BATCH = 128
CHUNKS = 80
EPW = CHUNKS * BATCH
E_PAD = NW * EPW

N_PAD = 10240
RPW = N_PAD // NS

_mesh = plsc.VectorSubcoreMesh(core_axis_name="c", subcore_axis_name="s")


@functools.partial(
    pl.kernel,
    out_type=jax.ShapeDtypeStruct((NC, N_PAD), jnp.float32),
    mesh=_mesh,
    scratch_types=[
        pltpu.VMEM((CHUNKS, BATCH), jnp.int32),
        pltpu.VMEM((BATCH,), jnp.float32),
        pltpu.VMEM((BATCH,), jnp.float32),
        pltpu.VMEM_SHARED((N_PAD,), jnp.float32),
    ],
)
def _deg_kernel(dst_hbm, degp_hbm, dst_v, ones_v, zero_v, deg_sh):
    c = lax.axis_index("c")
    s = lax.axis_index("s")
    wid = c * NS + s

    one = jnp.ones((16,), jnp.float32)
    z = jnp.zeros((16,), jnp.float32)
    for j in range(BATCH // 16):
        ones_v[pl.ds(j * 16, 16)] = one
        zero_v[pl.ds(j * 16, 16)] = z

    base = s * RPW
    for k in range(RPW // BATCH):
        pltpu.sync_copy(zero_v, deg_sh.at[pl.ds(base + k * BATCH, BATCH)])

    pltpu.sync_copy(dst_hbm.at[wid], dst_v)
    plsc.subcore_barrier()

    @pl.loop(0, CHUNKS)
    def _(j):
        pltpu.sync_copy(ones_v, deg_sh.at[dst_v.at[j]], add=True)

    plsc.subcore_barrier()
    pltpu.sync_copy(deg_sh.at[pl.ds(base, RPW)],
                    degp_hbm.at[c, pl.ds(base, RPW)])


NBUF = 4


@functools.partial(
    pl.kernel,
    out_type=jax.ShapeDtypeStruct((NC, N_PAD, C), jnp.float32),
    mesh=_mesh,
    scratch_types=[
        pltpu.VMEM((CHUNKS, BATCH), jnp.int32),
        pltpu.VMEM((CHUNKS, BATCH), jnp.int32),
        [pltpu.VMEM((BATCH, C), jnp.float32)] * NBUF,
        pltpu.VMEM_SHARED((N_PAD, C), jnp.float32),
        [pltpu.SemaphoreType.DMA] * NBUF,
        [pltpu.SemaphoreType.DMA] * NBUF,
    ],
    compiler_params=pltpu.CompilerParams(use_tc_tiling_on_sc=False),
)
def _agg_kernel(h2_hbm, src_hbm, dst_hbm, accp_hbm,
                src_v, dst_v, bufs, acc_sh, gsems, ssems):
    c = lax.axis_index("c")
    s = lax.axis_index("s")
    wid = c * NS + s
    base = s * RPW

    z = jnp.zeros((16,), jnp.float32)

    @pl.loop(0, BATCH)
    def _(i):
        for j in range(C // 16):
            bufs[0][i, pl.ds(j * 16, 16)] = z

    for k in range(RPW // BATCH):
        pltpu.sync_copy(bufs[0], acc_sh.at[pl.ds(base + k * BATCH, BATCH)])

    pltpu.sync_copy(src_hbm.at[wid], src_v)
    pltpu.sync_copy(dst_hbm.at[wid], dst_v)
    plsc.subcore_barrier()

    for b in range(NBUF):
        pltpu.async_copy(h2_hbm.at[src_v.at[b]], bufs[b], gsems[b])

    @pl.loop(0, CHUNKS, step=NBUF)
    def _(j):
        for b in range(NBUF):
            jj = j + b
            pltpu.make_async_copy(h2_hbm.at[src_v.at[jj]], bufs[b], gsems[b]).wait()
            pltpu.async_copy(bufs[b], acc_sh.at[dst_v.at[jj]], ssems[b], add=True)
        for b in range(NBUF):
            nxt = j + b + NBUF

            @pl.when(nxt < CHUNKS)
            def _():
                pltpu.make_async_copy(
                    bufs[b], acc_sh.at[dst_v.at[nxt - NBUF]], ssems[b]).wait()
                pltpu.async_copy(h2_hbm.at[src_v.at[nxt]], bufs[b], gsems[b])

    for b in range(NBUF):
        pltpu.make_async_copy(
            bufs[b], acc_sh.at[dst_v.at[CHUNKS - NBUF + b]], ssems[b]).wait()

    plsc.subcore_barrier()
    pltpu.sync_copy(acc_sh.at[pl.ds(base, RPW)],
                    accp_hbm.at[c, pl.ds(base, RPW)])


RB = 1024


def _h2_body(deg_ref, x_ref, w_ref, h2_ref):
    deg = deg_ref[0] + deg_ref[1] + 1.0
    dinv = lax.rsqrt(deg)
    h = jnp.dot(x_ref[...], w_ref[...], preferred_element_type=jnp.float32)
    h2_ref[...] = h * dinv[:, None]


def _fin_body(deg_ref, acc_ref, h2_ref, b_ref, out_ref):
    deg = deg_ref[0] + deg_ref[1] + 1.0
    dinv = lax.rsqrt(deg)
    tot = acc_ref[0] + acc_ref[1] + h2_ref[...]
    out_ref[...] = tot * dinv[:, None] + b_ref[...]


def _tc_h2(degp, x_pad, W):
    return pl.pallas_call(
        _h2_body,
        grid=(N_PAD // RB,),
        in_specs=[
            pl.BlockSpec((NC, RB), lambda i: (0, i)),
            pl.BlockSpec((RB, F), lambda i: (i, 0)),
            pl.BlockSpec((F, C), lambda i: (0, 0)),
        ],
        out_specs=pl.BlockSpec((RB, C), lambda i: (i, 0)),
        out_shape=jax.ShapeDtypeStruct((N_PAD, C), jnp.float32),
    )(degp, x_pad, W)


def _tc_final(degp, accp, h2, b):
    return pl.pallas_call(
        _fin_body,
        grid=(N_PAD // RB,),
        in_specs=[
            pl.BlockSpec((NC, RB), lambda i: (0, i)),
            pl.BlockSpec((NC, RB, C), lambda i: (0, i, 0)),
            pl.BlockSpec((RB, C), lambda i: (i, 0)),
            pl.BlockSpec((1, C), lambda i: (0, 0)),
        ],
        out_specs=pl.BlockSpec((RB, C), lambda i: (i, 0)),
        out_shape=jax.ShapeDtypeStruct((N_PAD, C), jnp.float32),
    )(degp, accp, h2, b)


def kernel(x, adj, W, b):
    src = adj[0].astype(jnp.int32)
    dst = adj[1].astype(jnp.int32)
    pad = E_PAD - E
    src3 = jnp.concatenate([src, jnp.zeros((pad,), jnp.int32)]).reshape(NW, CHUNKS, BATCH)
    dst3 = jnp.concatenate([dst, jnp.full((pad,), N, jnp.int32)]).reshape(NW, CHUNKS, BATCH)
    x_pad = jnp.concatenate([x, jnp.zeros((N_PAD - N, F), jnp.float32)])

    degp = _deg_kernel(dst3)
    h2 = _tc_h2(degp, x_pad, W)
    accp = _agg_kernel(h2, src3, dst3)
    out = _tc_final(degp, accp, h2, b.reshape(1, C))
    return out[:N]


# speedup vs baseline: 1.5930x; 1.5930x over previous
import functools
import jax
import jax.numpy as jnp
from jax import lax
from jax.experimental import pallas as pl
from jax.experimental.pallas import tpu as pltpu
from jax.experimental.pallas import tpu_sc as plsc

N = 10000
E = 320000
F = 128
C = 64
C2 = C // 2

NC = 2
NS = 16
NW = NC * NS

BATCH = 128
CHUNKS = 80
EPW = CHUNKS * BATCH
E_PAD = NW * EPW

N_PAD = 10240
RPW = N_PAD // NS

_mesh = plsc.VectorSubcoreMesh(core_axis_name="c", subcore_axis_name="s")


@functools.partial(
    pl.kernel,
    out_type=jax.ShapeDtypeStruct((NC, N_PAD), jnp.float32),
    mesh=_mesh,
    scratch_types=[
        pltpu.VMEM((CHUNKS, BATCH), jnp.int32),
        pltpu.VMEM((BATCH,), jnp.float32),
        pltpu.VMEM((BATCH,), jnp.float32),
        pltpu.VMEM_SHARED((N_PAD,), jnp.float32),
    ],
)
def _deg_kernel(dst_hbm, degp_hbm, dst_v, ones_v, zero_v, deg_sh):
    c = lax.axis_index("c")
    s = lax.axis_index("s")
    wid = c * NS + s

    one = jnp.ones((16,), jnp.float32)
    z = jnp.zeros((16,), jnp.float32)
    for j in range(BATCH // 16):
        ones_v[pl.ds(j * 16, 16)] = one
        zero_v[pl.ds(j * 16, 16)] = z

    base = s * RPW
    for k in range(RPW // BATCH):
        pltpu.sync_copy(zero_v, deg_sh.at[pl.ds(base + k * BATCH, BATCH)])

    pltpu.sync_copy(dst_hbm.at[wid], dst_v)
    plsc.subcore_barrier()

    @pl.loop(0, CHUNKS)
    def _(j):
        pltpu.sync_copy(ones_v, deg_sh.at[dst_v.at[j]], add=True)

    plsc.subcore_barrier()
    pltpu.sync_copy(deg_sh.at[pl.ds(base, RPW)],
                    degp_hbm.at[c, pl.ds(base, RPW)])


NBUF = 4


@functools.partial(
    pl.kernel,
    out_type=jax.ShapeDtypeStruct((NC, 2, N_PAD, C2), jnp.float32),
    mesh=_mesh,
    scratch_types=[
        pltpu.VMEM((CHUNKS, BATCH), jnp.int32),
        pltpu.VMEM((CHUNKS, BATCH), jnp.int32),
        [pltpu.VMEM((BATCH, C2), jnp.float32)] * NBUF,
        pltpu.VMEM_SHARED((N_PAD, C2), jnp.float32),
        pltpu.VMEM_SHARED((N_PAD, C2), jnp.float32),
        [pltpu.SemaphoreType.DMA] * NBUF,
        [pltpu.SemaphoreType.DMA] * NBUF,
    ],
    compiler_params=pltpu.CompilerParams(use_tc_tiling_on_sc=False),
)
def _agg_kernel(h2s_hbm, src_hbm, dst_hbm, accp_hbm,
                src_v, dst_v, bufs, h2_sh, acc_sh, gsems, ssems):
    c = lax.axis_index("c")
    s = lax.axis_index("s")
    wid = c * NS + s
    base = s * RPW

    z = jnp.zeros((16,), jnp.float32)

    @pl.loop(0, BATCH)
    def _(i):
        for j in range(C2 // 16):
            bufs[0][i, pl.ds(j * 16, 16)] = z

    pltpu.sync_copy(src_hbm.at[wid], src_v)
    pltpu.sync_copy(dst_hbm.at[wid], dst_v)

    for p in range(2):
        pltpu.sync_copy(h2s_hbm.at[p, pl.ds(base, RPW)],
                        h2_sh.at[pl.ds(base, RPW)])
        for k in range(RPW // BATCH):
            pltpu.sync_copy(bufs[0], acc_sh.at[pl.ds(base + k * BATCH, BATCH)])
        plsc.subcore_barrier()

        for b in range(NBUF):
            pltpu.async_copy(h2_sh.at[src_v.at[b]], bufs[b], gsems[b])

        @pl.loop(0, CHUNKS, step=NBUF)
        def _(j):
            for b in range(NBUF):
                jj = j + b
                pltpu.make_async_copy(h2_sh.at[src_v.at[jj]], bufs[b], gsems[b]).wait()
                pltpu.async_copy(bufs[b], acc_sh.at[dst_v.at[jj]], ssems[b], add=True)
            for b in range(NBUF):
                nxt = j + b + NBUF

                @pl.when(nxt < CHUNKS)
                def _():
                    pltpu.make_async_copy(
                        bufs[b], acc_sh.at[dst_v.at[nxt - NBUF]], ssems[b]).wait()
                    pltpu.async_copy(h2_sh.at[src_v.at[nxt]], bufs[b], gsems[b])

        for b in range(NBUF):
            pltpu.make_async_copy(
                bufs[b], acc_sh.at[dst_v.at[CHUNKS - NBUF + b]], ssems[b]).wait()

        plsc.subcore_barrier()
        pltpu.sync_copy(acc_sh.at[pl.ds(base, RPW)],
                        accp_hbm.at[c, p, pl.ds(base, RPW)])
        if p == 0:
            @pl.loop(0, BATCH)
            def _(i):
                for j in range(C2 // 16):
                    bufs[0][i, pl.ds(j * 16, 16)] = z


RB = 1024


def _h2_body(deg_ref, x_ref, w_ref, h2s_ref):
    deg = deg_ref[0] + deg_ref[1] + 1.0
    dinv = lax.rsqrt(deg)
    h = jnp.dot(x_ref[...], w_ref[...], preferred_element_type=jnp.float32)
    h2 = h * dinv[:, None]
    h2s_ref[0] = h2[:, :C2]
    h2s_ref[1] = h2[:, C2:]


def _fin_body(deg_ref, acc_ref, h2s_ref, b_ref, out_ref):
    deg = deg_ref[0] + deg_ref[1] + 1.0
    dinv = lax.rsqrt(deg)
    lo = (acc_ref[0, 0] + acc_ref[1, 0] + h2s_ref[0]) * dinv[:, None]
    hi = (acc_ref[0, 1] + acc_ref[1, 1] + h2s_ref[1]) * dinv[:, None]
    out_ref[...] = jnp.concatenate([lo, hi], axis=1) + b_ref[...]


def _tc_h2(degp, x_pad, W):
    return pl.pallas_call(
        _h2_body,
        grid=(N_PAD // RB,),
        in_specs=[
            pl.BlockSpec((NC, RB), lambda i: (0, i)),
            pl.BlockSpec((RB, F), lambda i: (i, 0)),
            pl.BlockSpec((F, C), lambda i: (0, 0)),
        ],
        out_specs=pl.BlockSpec((2, RB, C2), lambda i: (0, i, 0)),
        out_shape=jax.ShapeDtypeStruct((2, N_PAD, C2), jnp.float32),
    )(degp, x_pad, W)


def _tc_final(degp, accp, h2s, b):
    return pl.pallas_call(
        _fin_body,
        grid=(N_PAD // RB,),
        in_specs=[
            pl.BlockSpec((NC, RB), lambda i: (0, i)),
            pl.BlockSpec((NC, 2, RB, C2), lambda i: (0, 0, i, 0)),
            pl.BlockSpec((2, RB, C2), lambda i: (0, i, 0)),
            pl.BlockSpec((1, C), lambda i: (0, 0)),
        ],
        out_specs=pl.BlockSpec((RB, C), lambda i: (i, 0)),
        out_shape=jax.ShapeDtypeStruct((N_PAD, C), jnp.float32),
    )(degp, accp, h2s, b)


def kernel(x, adj, W, b):
    src = adj[0].astype(jnp.int32)
    dst = adj[1].astype(jnp.int32)
    pad = E_PAD - E
    src3 = jnp.concatenate([src, jnp.zeros((pad,), jnp.int32)]).reshape(NW, CHUNKS, BATCH)
    dst3 = jnp.concatenate([dst, jnp.full((pad,), N, jnp.int32)]).reshape(NW, CHUNKS, BATCH)
    x_pad = jnp.concatenate([x, jnp.zeros((N_PAD - N, F), jnp.float32)])

    degp = _deg_kernel(dst3)
    h2s = _tc_h2(degp, x_pad, W)
    accp = _agg_kernel(h2s, src3, dst3)
    out = _tc_final(degp, accp, h2s, b.reshape(1, C))
    return out[:N]


# speedup vs baseline: 1.7840x; 1.1199x over previous
import functools
import jax
import jax.numpy as jnp
from jax import lax
from jax.experimental import pallas as pl
from jax.experimental.pallas import tpu as pltpu
from jax.experimental.pallas import tpu_sc as plsc

N = 10000
E = 320000
F = 128
C = 64
C2 = C // 2

NC = 2
NS = 16
NW = NC * NS

BATCH = 128
NCHUNK = E // BATCH
CPW = NCHUNK // NW
XTRA = NCHUNK % NW

N_PAD = 10240
RPW = N_PAD // NS

_mesh = plsc.VectorSubcoreMesh(core_axis_name="c", subcore_axis_name="s")


def _chunk_range(wid):
    extra = (wid < XTRA).astype(jnp.int32)
    off = wid * CPW + jnp.minimum(wid, XTRA)
    return off, CPW + extra


@functools.partial(
    pl.kernel,
    out_type=jax.ShapeDtypeStruct((NC, N_PAD), jnp.float32),
    mesh=_mesh,
    scratch_types=[
        pltpu.VMEM((CPW + 1, BATCH), jnp.int32),
        pltpu.VMEM((BATCH,), jnp.float32),
        pltpu.VMEM((BATCH,), jnp.float32),
        pltpu.VMEM_SHARED((N_PAD,), jnp.float32),
    ],
    compiler_params=pltpu.CompilerParams(use_tc_tiling_on_sc=False),
)
def _deg_kernel(adj_hbm, degp_hbm, dst_v, ones_v, zero_v, deg_sh):
    c = lax.axis_index("c")
    s = lax.axis_index("s")
    wid = c * NS + s
    off, nch = _chunk_range(wid)

    one = jnp.ones((16,), jnp.float32)
    z = jnp.zeros((16,), jnp.float32)
    for j in range(BATCH // 16):
        ones_v[pl.ds(j * 16, 16)] = one
        zero_v[pl.ds(j * 16, 16)] = z

    base = s * RPW
    for k in range(RPW // BATCH):
        pltpu.sync_copy(zero_v, deg_sh.at[pl.ds(base + k * BATCH, BATCH)])

    @pl.when(wid < XTRA)
    def _():
        pltpu.sync_copy(adj_hbm.at[1, pl.ds(off, CPW + 1)], dst_v)

    @pl.when(wid >= XTRA)
    def _():
        pltpu.sync_copy(adj_hbm.at[1, pl.ds(off, CPW)], dst_v.at[pl.ds(0, CPW)])

    plsc.subcore_barrier()

    @pl.loop(0, nch)
    def _(j):
        pltpu.sync_copy(ones_v, deg_sh.at[dst_v.at[j]], add=True)

    plsc.subcore_barrier()
    pltpu.sync_copy(deg_sh.at[pl.ds(base, RPW)],
                    degp_hbm.at[c, pl.ds(base, RPW)])


NBUF = 2


@functools.partial(
    pl.kernel,
    out_type=jax.ShapeDtypeStruct((NC, 2, N_PAD, C2), jnp.float32),
    mesh=_mesh,
    scratch_types=[
        pltpu.VMEM((CPW + 1, BATCH), jnp.int32),
        pltpu.VMEM((CPW + 1, BATCH), jnp.int32),
        [pltpu.VMEM((BATCH, C2), jnp.float32)] * NBUF,
        pltpu.VMEM((BATCH, C2), jnp.float32),
        pltpu.VMEM_SHARED((N_PAD, C2), jnp.float32),
        pltpu.VMEM_SHARED((N_PAD, C2), jnp.float32),
        [pltpu.SemaphoreType.DMA] * NBUF,
        [pltpu.SemaphoreType.DMA] * NBUF,
    ],
    compiler_params=pltpu.CompilerParams(use_tc_tiling_on_sc=False),
)
def _agg_kernel(h2s_hbm, adj_hbm, accp_hbm,
                src_v, dst_v, bufs, zbuf, h2_sh, acc_sh, gsems, ssems):
    c = lax.axis_index("c")
    s = lax.axis_index("s")
    wid = c * NS + s
    off, nch = _chunk_range(wid)
    base = s * RPW

    z = jnp.zeros((16,), jnp.float32)

    @pl.loop(0, BATCH)
    def _(i):
        for j in range(C2 // 16):
            zbuf[i, pl.ds(j * 16, 16)] = z

    @pl.when(wid < XTRA)
    def _():
        pltpu.sync_copy(adj_hbm.at[0, pl.ds(off, CPW + 1)], src_v)
        pltpu.sync_copy(adj_hbm.at[1, pl.ds(off, CPW + 1)], dst_v)

    @pl.when(wid >= XTRA)
    def _():
        pltpu.sync_copy(adj_hbm.at[0, pl.ds(off, CPW)], src_v.at[pl.ds(0, CPW)])
        pltpu.sync_copy(adj_hbm.at[1, pl.ds(off, CPW)], dst_v.at[pl.ds(0, CPW)])

    for p in range(2):
        pltpu.sync_copy(h2s_hbm.at[p, pl.ds(base, RPW)],
                        h2_sh.at[pl.ds(base, RPW)])
        for k in range(RPW // BATCH):
            pltpu.sync_copy(zbuf, acc_sh.at[pl.ds(base + k * BATCH, BATCH)])
        plsc.subcore_barrier()

        for b in range(NBUF):
            pltpu.async_copy(h2_sh.at[src_v.at[b]], bufs[b], gsems[b])

        @pl.loop(0, CPW, step=NBUF)
        def _(j):
            for b in range(NBUF):
                jj = j + b
                pltpu.make_async_copy(h2_sh.at[src_v.at[jj]], bufs[b], gsems[b]).wait()
                pltpu.async_copy(bufs[b], acc_sh.at[dst_v.at[jj]], ssems[b], add=True)
            for b in range(NBUF):
                nxt = j + b + NBUF

                @pl.when(nxt < CPW)
                def _():
                    pltpu.make_async_copy(
                        bufs[b], acc_sh.at[dst_v.at[nxt - NBUF]], ssems[b]).wait()
                    pltpu.async_copy(h2_sh.at[src_v.at[nxt]], bufs[b], gsems[b])

        for b in range(NBUF):
            pltpu.make_async_copy(
                bufs[b], acc_sh.at[dst_v.at[CPW - NBUF + b]], ssems[b]).wait()

        @pl.when(nch > CPW)
        def _():
            pltpu.sync_copy(h2_sh.at[src_v.at[CPW]], zbuf)
            pltpu.sync_copy(zbuf, acc_sh.at[dst_v.at[CPW]], add=True)

        plsc.subcore_barrier()
        pltpu.sync_copy(acc_sh.at[pl.ds(base, RPW)],
                        accp_hbm.at[c, p, pl.ds(base, RPW)])

        if p == 0:
            @pl.when(nch > CPW)
            def _():
                @pl.loop(0, BATCH)
                def _(i):
                    for j in range(C2 // 16):
                        zbuf[i, pl.ds(j * 16, 16)] = z


RB = 1024


def _h2_body(deg_ref, x_ref, w_ref, h2s_ref):
    deg = deg_ref[0] + deg_ref[1] + 1.0
    dinv = lax.rsqrt(deg)
    h = jnp.dot(x_ref[...], w_ref[...], preferred_element_type=jnp.float32)
    h2 = h * dinv[:, None]
    h2s_ref[0] = h2[:, :C2]
    h2s_ref[1] = h2[:, C2:]


def _fin_body(deg_ref, acc_ref, h2s_ref, b_ref, out_ref):
    deg = deg_ref[0] + deg_ref[1] + 1.0
    dinv = lax.rsqrt(deg)
    lo = (acc_ref[0, 0] + acc_ref[1, 0] + h2s_ref[0]) * dinv[:, None]
    hi = (acc_ref[0, 1] + acc_ref[1, 1] + h2s_ref[1]) * dinv[:, None]
    out_ref[...] = jnp.concatenate([lo, hi], axis=1) + b_ref[...]


def _tc_h2(degp, x, W):
    return pl.pallas_call(
        _h2_body,
        grid=(N_PAD // RB,),
        in_specs=[
            pl.BlockSpec((NC, RB), lambda i: (0, i)),
            pl.BlockSpec((RB, F), lambda i: (i, 0)),
            pl.BlockSpec((F, C), lambda i: (0, 0)),
        ],
        out_specs=pl.BlockSpec((2, RB, C2), lambda i: (0, i, 0)),
        out_shape=jax.ShapeDtypeStruct((2, N_PAD, C2), jnp.float32),
    )(degp, x, W)


def _tc_final(degp, accp, h2s, b):
    return pl.pallas_call(
        _fin_body,
        grid=(N_PAD // RB,),
        in_specs=[
            pl.BlockSpec((NC, RB), lambda i: (0, i)),
            pl.BlockSpec((NC, 2, RB, C2), lambda i: (0, 0, i, 0)),
            pl.BlockSpec((2, RB, C2), lambda i: (0, i, 0)),
            pl.BlockSpec((1, C), lambda i: (0, 0)),
        ],
        out_specs=pl.BlockSpec((RB, C), lambda i: (i, 0)),
        out_shape=jax.ShapeDtypeStruct((N_PAD, C), jnp.float32),
    )(degp, accp, h2s, b)


def kernel(x, adj, W, b):
    adj2d = adj.astype(jnp.int32).reshape(2, NCHUNK, BATCH)

    degp = _deg_kernel(adj2d)
    h2s = _tc_h2(degp, x, W)
    accp = _agg_kernel(h2s, adj2d)
    out = _tc_final(degp, accp, h2s, b.reshape(1, C))
    return out[:N]


# speedup vs baseline: 2.0744x; 1.1628x over previous
import functools
import jax
import jax.numpy as jnp
from jax import lax
from jax.experimental import pallas as pl
from jax.experimental.pallas import tpu as pltpu
from jax.experimental.pallas import tpu_sc as plsc

N = 10000
E = 320000
F = 128
C = 64
C2 = C // 2

NC = 2
NS = 16
NW = NC * NS

BATCH = 128
NCHUNK = E // BATCH
CPW = NCHUNK // NW
XTRA = NCHUNK % NW

N_PAD = 10240
RPW = N_PAD // NS

_mesh = plsc.VectorSubcoreMesh(core_axis_name="c", subcore_axis_name="s")


def _chunk_range(wid):
    extra = (wid < XTRA).astype(jnp.int32)
    off = wid * CPW + jnp.minimum(wid, XTRA)
    return off, CPW + extra


@functools.partial(
    pl.kernel,
    out_type=jax.ShapeDtypeStruct((NC, N_PAD), jnp.float32),
    mesh=_mesh,
    scratch_types=[
        pltpu.VMEM((CPW + 1, BATCH), jnp.int32),
        pltpu.VMEM((BATCH,), jnp.float32),
        pltpu.VMEM((BATCH,), jnp.float32),
        pltpu.VMEM_SHARED((N_PAD,), jnp.float32),
    ],
    compiler_params=pltpu.CompilerParams(use_tc_tiling_on_sc=False),
)
def _deg_kernel(adj_hbm, degp_hbm, dst_v, ones_v, zero_v, deg_sh):
    c = lax.axis_index("c")
    s = lax.axis_index("s")
    wid = c * NS + s
    off, nch = _chunk_range(wid)

    one = jnp.ones((16,), jnp.float32)
    z = jnp.zeros((16,), jnp.float32)
    for j in range(BATCH // 16):
        ones_v[pl.ds(j * 16, 16)] = one
        zero_v[pl.ds(j * 16, 16)] = z

    base = s * RPW
    for k in range(RPW // BATCH):
        pltpu.sync_copy(zero_v, deg_sh.at[pl.ds(base + k * BATCH, BATCH)])

    @pl.when(wid < XTRA)
    def _():
        pltpu.sync_copy(adj_hbm.at[1, pl.ds(off, CPW + 1)], dst_v)

    @pl.when(wid >= XTRA)
    def _():
        pltpu.sync_copy(adj_hbm.at[1, pl.ds(off, CPW)], dst_v.at[pl.ds(0, CPW)])

    plsc.subcore_barrier()

    @pl.loop(0, nch)
    def _(j):
        pltpu.sync_copy(ones_v, deg_sh.at[dst_v.at[j]], add=True)

    plsc.subcore_barrier()
    pltpu.sync_copy(deg_sh.at[pl.ds(base, RPW)],
                    degp_hbm.at[c, pl.ds(base, RPW)])


NBUF = 2


@functools.partial(
    pl.kernel,
    out_type=jax.ShapeDtypeStruct((NC, N_PAD, 128), jnp.float32),
    mesh=_mesh,
    scratch_types=[
        pltpu.VMEM((CPW + 1, BATCH), jnp.int32),
        pltpu.VMEM((CPW + 1, BATCH), jnp.int32),
        [pltpu.VMEM((BATCH, C2), jnp.float32)] * NBUF,
        pltpu.VMEM((BATCH, C2), jnp.float32),
        pltpu.VMEM_SHARED((N_PAD, C2), jnp.float32),
        pltpu.VMEM_SHARED((N_PAD, C2), jnp.float32),
        [pltpu.SemaphoreType.DMA] * NBUF,
        [pltpu.SemaphoreType.DMA] * NBUF,
    ],
    compiler_params=pltpu.CompilerParams(use_tc_tiling_on_sc=False),
)
def _agg_kernel(h2s_hbm, adj_hbm, accp_hbm,
                src_v, dst_v, bufs, zbuf, h2_sh, acc_sh, gsems, ssems):
    c = lax.axis_index("c")
    s = lax.axis_index("s")
    wid = c * NS + s
    off, nch = _chunk_range(wid)
    base = s * RPW

    z = jnp.zeros((16,), jnp.float32)

    @pl.loop(0, BATCH)
    def _(i):
        for j in range(C2 // 16):
            zbuf[i, pl.ds(j * 16, 16)] = z

    @pl.when(wid < XTRA)
    def _():
        pltpu.sync_copy(adj_hbm.at[0, pl.ds(off, CPW + 1)], src_v)
        pltpu.sync_copy(adj_hbm.at[1, pl.ds(off, CPW + 1)], dst_v)

    @pl.when(wid >= XTRA)
    def _():
        pltpu.sync_copy(adj_hbm.at[0, pl.ds(off, CPW)], src_v.at[pl.ds(0, CPW)])
        pltpu.sync_copy(adj_hbm.at[1, pl.ds(off, CPW)], dst_v.at[pl.ds(0, CPW)])

    for p in range(2):
        pltpu.sync_copy(h2s_hbm.at[pl.ds(base, RPW), pl.ds(p * C2, C2)],
                        h2_sh.at[pl.ds(base, RPW)])
        for k in range(RPW // BATCH):
            pltpu.sync_copy(zbuf, acc_sh.at[pl.ds(base + k * BATCH, BATCH)])
        plsc.subcore_barrier()

        for b in range(NBUF):
            pltpu.async_copy(h2_sh.at[src_v.at[b]], bufs[b], gsems[b])

        @pl.loop(0, CPW, step=NBUF)
        def _(j):
            for b in range(NBUF):
                jj = j + b
                pltpu.make_async_copy(h2_sh.at[src_v.at[jj]], bufs[b], gsems[b]).wait()
                pltpu.async_copy(bufs[b], acc_sh.at[dst_v.at[jj]], ssems[b], add=True)
            for b in range(NBUF):
                nxt = j + b + NBUF

                @pl.when(nxt < CPW)
                def _():
                    pltpu.make_async_copy(
                        bufs[b], acc_sh.at[dst_v.at[nxt - NBUF]], ssems[b]).wait()
                    pltpu.async_copy(h2_sh.at[src_v.at[nxt]], bufs[b], gsems[b])

        for b in range(NBUF):
            pltpu.make_async_copy(
                bufs[b], acc_sh.at[dst_v.at[CPW - NBUF + b]], ssems[b]).wait()

        @pl.when(nch > CPW)
        def _():
            pltpu.sync_copy(h2_sh.at[src_v.at[CPW]], zbuf)
            pltpu.sync_copy(zbuf, acc_sh.at[dst_v.at[CPW]], add=True)

        plsc.subcore_barrier()
        pltpu.sync_copy(acc_sh.at[pl.ds(base, RPW)],
                        accp_hbm.at[c, pl.ds(base, RPW), pl.ds(p * C2, C2)])

        if p == 0:
            @pl.when(nch > CPW)
            def _():
                @pl.loop(0, BATCH)
                def _(i):
                    for j in range(C2 // 16):
                        zbuf[i, pl.ds(j * 16, 16)] = z


RB = 1024


def _h2_body(deg_ref, x_ref, w_ref, h2f_ref):
    deg = deg_ref[0] + deg_ref[1] + 1.0
    dinv = lax.rsqrt(deg)
    h = jnp.dot(x_ref[...], w_ref[...], preferred_element_type=jnp.float32)
    h2 = h * dinv[:, None]
    h2f_ref[...] = jnp.concatenate([h2, h2], axis=1)


def _fin_body(deg_ref, acc_ref, h2f_ref, b_ref, out_ref):
    deg = deg_ref[0] + deg_ref[1] + 1.0
    dinv = lax.rsqrt(deg)
    tot = acc_ref[0, :, :C] + acc_ref[1, :, :C] + h2f_ref[:, :C]
    out_ref[...] = tot * dinv[:, None] + b_ref[...]


def _tc_h2(degp, x, W):
    return pl.pallas_call(
        _h2_body,
        grid=(N_PAD // RB,),
        in_specs=[
            pl.BlockSpec((NC, RB), lambda i: (0, i)),
            pl.BlockSpec((RB, F), lambda i: (i, 0)),
            pl.BlockSpec((F, C), lambda i: (0, 0)),
        ],
        out_specs=pl.BlockSpec((RB, 128), lambda i: (i, 0)),
        out_shape=jax.ShapeDtypeStruct((N_PAD, 128), jnp.float32),
    )(degp, x, W)


def _tc_final(degp, accp, h2s, b):
    return pl.pallas_call(
        _fin_body,
        grid=(N_PAD // RB,),
        in_specs=[
            pl.BlockSpec((NC, RB), lambda i: (0, i)),
            pl.BlockSpec((NC, RB, 128), lambda i: (0, i, 0)),
            pl.BlockSpec((RB, 128), lambda i: (i, 0)),
            pl.BlockSpec((1, C), lambda i: (0, 0)),
        ],
        out_specs=pl.BlockSpec((RB, C), lambda i: (i, 0)),
        out_shape=jax.ShapeDtypeStruct((N, C), jnp.float32),
    )(degp, accp, h2s, b)


def kernel(x, adj, W, b):
    adj2d = adj.astype(jnp.int32).reshape(2, NCHUNK, BATCH)

    degp = _deg_kernel(adj2d)
    h2s = _tc_h2(degp, x, W)
    accp = _agg_kernel(h2s, adj2d)
    return _tc_final(degp, accp, h2s, b.reshape(1, C))


# speedup vs baseline: 2.0843x; 1.0048x over previous
import functools
import jax
import jax.numpy as jnp
from jax import lax
from jax.experimental import pallas as pl
from jax.experimental.pallas import tpu as pltpu
from jax.experimental.pallas import tpu_sc as plsc

N = 10000
E = 320000
F = 128
C = 64
C2 = C // 2

NC = 2
NS = 16
NW = NC * NS

BATCH = 128
NCHUNK = E // BATCH
CPW = NCHUNK // NW
XTRA = NCHUNK % NW

N_PAD = 10240
RPW = N_PAD // NS

_mesh = plsc.VectorSubcoreMesh(core_axis_name="c", subcore_axis_name="s")


def _chunk_range(wid):
    extra = (wid < XTRA).astype(jnp.int32)
    off = wid * CPW + jnp.minimum(wid, XTRA)
    return off, CPW + extra


@functools.partial(
    pl.kernel,
    out_type=jax.ShapeDtypeStruct((NC, N_PAD), jnp.float32),
    mesh=_mesh,
    scratch_types=[
        pltpu.VMEM((CPW + 1, BATCH), jnp.int32),
        pltpu.VMEM((BATCH,), jnp.float32),
        pltpu.VMEM((BATCH,), jnp.float32),
        pltpu.VMEM_SHARED((N_PAD,), jnp.float32),
    ],
    compiler_params=pltpu.CompilerParams(use_tc_tiling_on_sc=False),
)
def _deg_kernel(adj_hbm, degp_hbm, dst_v, ones_v, zero_v, deg_sh):
    c = lax.axis_index("c")
    s = lax.axis_index("s")
    wid = c * NS + s
    off, nch = _chunk_range(wid)

    one = jnp.ones((16,), jnp.float32)
    z = jnp.zeros((16,), jnp.float32)
    for j in range(BATCH // 16):
        ones_v[pl.ds(j * 16, 16)] = one
        zero_v[pl.ds(j * 16, 16)] = z

    base = s * RPW
    for k in range(RPW // BATCH):
        pltpu.sync_copy(zero_v, deg_sh.at[pl.ds(base + k * BATCH, BATCH)])

    @pl.when(wid < XTRA)
    def _():
        pltpu.sync_copy(adj_hbm.at[1, pl.ds(off, CPW + 1)], dst_v)

    @pl.when(wid >= XTRA)
    def _():
        pltpu.sync_copy(adj_hbm.at[1, pl.ds(off, CPW)], dst_v.at[pl.ds(0, CPW)])

    plsc.subcore_barrier()

    @pl.loop(0, nch)
    def _(j):
        pltpu.sync_copy(ones_v, deg_sh.at[dst_v.at[j]], add=True)

    plsc.subcore_barrier()
    pltpu.sync_copy(deg_sh.at[pl.ds(base, RPW)],
                    degp_hbm.at[c, pl.ds(base, RPW)])


NBUF = 2


@functools.partial(
    pl.kernel,
    out_type=jax.ShapeDtypeStruct((NC, N_PAD, 128), jnp.float32),
    mesh=_mesh,
    scratch_types=[
        pltpu.VMEM((CPW + 1, BATCH), jnp.int32),
        pltpu.VMEM((CPW + 1, BATCH), jnp.int32),
        [pltpu.VMEM((BATCH, C2), jnp.float32)] * NBUF,
        pltpu.VMEM((BATCH, C2), jnp.float32),
        [pltpu.VMEM_SHARED((N_PAD, C2), jnp.float32)] * 2,
        pltpu.VMEM_SHARED((N_PAD, C2), jnp.float32),
        [pltpu.SemaphoreType.DMA] * NBUF,
        [pltpu.SemaphoreType.DMA] * NBUF,
        pltpu.SemaphoreType.DMA,
    ],
    compiler_params=pltpu.CompilerParams(use_tc_tiling_on_sc=False),
)
def _agg_kernel(h2s_hbm, adj_hbm, accp_hbm,
                src_v, dst_v, bufs, zbuf, h2_shs, acc_sh, gsems, ssems, psem):
    c = lax.axis_index("c")
    s = lax.axis_index("s")
    wid = c * NS + s
    off, nch = _chunk_range(wid)
    base = s * RPW

    z = jnp.zeros((16,), jnp.float32)

    @pl.loop(0, BATCH)
    def _(i):
        for j in range(C2 // 16):
            zbuf[i, pl.ds(j * 16, 16)] = z

    @pl.when(wid < XTRA)
    def _():
        pltpu.sync_copy(adj_hbm.at[0, pl.ds(off, CPW + 1)], src_v)
        pltpu.sync_copy(adj_hbm.at[1, pl.ds(off, CPW + 1)], dst_v)

    @pl.when(wid >= XTRA)
    def _():
        pltpu.sync_copy(adj_hbm.at[0, pl.ds(off, CPW)], src_v.at[pl.ds(0, CPW)])
        pltpu.sync_copy(adj_hbm.at[1, pl.ds(off, CPW)], dst_v.at[pl.ds(0, CPW)])

    pltpu.sync_copy(h2s_hbm.at[pl.ds(base, RPW), pl.ds(0, C2)],
                    h2_shs[0].at[pl.ds(base, RPW)])
    pltpu.async_copy(h2s_hbm.at[pl.ds(base, RPW), pl.ds(C2, C2)],
                     h2_shs[1].at[pl.ds(base, RPW)], psem)

    for p in range(2):
        h2_sh = h2_shs[p]
        if p == 1:
            pltpu.make_async_copy(h2s_hbm.at[pl.ds(base, RPW), pl.ds(C2, C2)],
                                  h2_shs[1].at[pl.ds(base, RPW)], psem).wait()
        for k in range(RPW // BATCH):
            pltpu.sync_copy(zbuf, acc_sh.at[pl.ds(base + k * BATCH, BATCH)])
        plsc.subcore_barrier()

        for b in range(NBUF):
            pltpu.async_copy(h2_sh.at[src_v.at[b]], bufs[b], gsems[b])

        @pl.loop(0, CPW, step=NBUF)
        def _(j):
            for b in range(NBUF):
                jj = j + b
                pltpu.make_async_copy(h2_sh.at[src_v.at[jj]], bufs[b], gsems[b]).wait()
                pltpu.async_copy(bufs[b], acc_sh.at[dst_v.at[jj]], ssems[b], add=True)
            for b in range(NBUF):
                nxt = j + b + NBUF

                @pl.when(nxt < CPW)
                def _():
                    pltpu.make_async_copy(
                        bufs[b], acc_sh.at[dst_v.at[nxt - NBUF]], ssems[b]).wait()
                    pltpu.async_copy(h2_sh.at[src_v.at[nxt]], bufs[b], gsems[b])

        for b in range(NBUF):
            pltpu.make_async_copy(
                bufs[b], acc_sh.at[dst_v.at[CPW - NBUF + b]], ssems[b]).wait()

        @pl.when(nch > CPW)
        def _():
            pltpu.sync_copy(h2_sh.at[src_v.at[CPW]], zbuf)
            pltpu.sync_copy(zbuf, acc_sh.at[dst_v.at[CPW]], add=True)

        plsc.subcore_barrier()
        pltpu.sync_copy(acc_sh.at[pl.ds(base, RPW)],
                        accp_hbm.at[c, pl.ds(base, RPW), pl.ds(p * C2, C2)])

        if p == 0:
            @pl.when(nch > CPW)
            def _():
                @pl.loop(0, BATCH)
                def _(i):
                    for j in range(C2 // 16):
                        zbuf[i, pl.ds(j * 16, 16)] = z


RB = 1024


def _mm_body(x_ref, w_ref, h_ref):
    h = jnp.dot(x_ref[...], w_ref[...], preferred_element_type=jnp.float32)
    h_ref[...] = jnp.concatenate([h, h], axis=1)


def _scale_body(deg_ref, h_ref, h2f_ref):
    deg = deg_ref[0] + deg_ref[1] + 1.0
    dinv = lax.rsqrt(deg)
    h2f_ref[...] = h_ref[...] * dinv[:, None]


def _fin_body(deg_ref, acc_ref, h2f_ref, b_ref, out_ref):
    deg = deg_ref[0] + deg_ref[1] + 1.0
    dinv = lax.rsqrt(deg)
    tot = acc_ref[0, :, :C] + acc_ref[1, :, :C] + h2f_ref[:, :C]
    out_ref[...] = tot * dinv[:, None] + b_ref[...]


def _tc_mm(x, W):
    return pl.pallas_call(
        _mm_body,
        grid=(N_PAD // RB,),
        in_specs=[
            pl.BlockSpec((RB, F), lambda i: (i, 0)),
            pl.BlockSpec((F, C), lambda i: (0, 0)),
        ],
        out_specs=pl.BlockSpec((RB, 128), lambda i: (i, 0)),
        out_shape=jax.ShapeDtypeStruct((N_PAD, 128), jnp.float32),
    )(x, W)


def _tc_scale(degp, h):
    return pl.pallas_call(
        _scale_body,
        grid=(N_PAD // RB,),
        in_specs=[
            pl.BlockSpec((NC, RB), lambda i: (0, i)),
            pl.BlockSpec((RB, 128), lambda i: (i, 0)),
        ],
        out_specs=pl.BlockSpec((RB, 128), lambda i: (i, 0)),
        out_shape=jax.ShapeDtypeStruct((N_PAD, 128), jnp.float32),
    )(degp, h)


def _tc_final(degp, accp, h2s, b):
    return pl.pallas_call(
        _fin_body,
        grid=(N_PAD // RB,),
        in_specs=[
            pl.BlockSpec((NC, RB), lambda i: (0, i)),
            pl.BlockSpec((NC, RB, 128), lambda i: (0, i, 0)),
            pl.BlockSpec((RB, 128), lambda i: (i, 0)),
            pl.BlockSpec((1, C), lambda i: (0, 0)),
        ],
        out_specs=pl.BlockSpec((RB, C), lambda i: (i, 0)),
        out_shape=jax.ShapeDtypeStruct((N, C), jnp.float32),
    )(degp, accp, h2s, b)


def kernel(x, adj, W, b):
    adj2d = adj.astype(jnp.int32).reshape(2, NCHUNK, BATCH)

    degp = _deg_kernel(adj2d)
    h = _tc_mm(x, W)
    h2s = _tc_scale(degp, h)
    accp = _agg_kernel(h2s, adj2d)
    return _tc_final(degp, accp, h2s, b.reshape(1, C))


# speedup vs baseline: 2.0957x; 1.0055x over previous
import functools
import jax
import jax.numpy as jnp
from jax import lax
from jax.experimental import pallas as pl
from jax.experimental.pallas import tpu as pltpu
from jax.experimental.pallas import tpu_sc as plsc

N = 10000
E = 320000
F = 128
C = 64
C2 = C // 2

NC = 2
NS = 16
NW = NC * NS

BATCH = 128
NCHUNK = E // BATCH
CPW = NCHUNK // NW
XTRA = NCHUNK % NW

N_PAD = 10240
RPW = N_PAD // NS

_mesh = plsc.VectorSubcoreMesh(core_axis_name="c", subcore_axis_name="s")


def _chunk_range(wid):
    extra = (wid < XTRA).astype(jnp.int32)
    off = wid * CPW + jnp.minimum(wid, XTRA)
    return off, CPW + extra


@functools.partial(
    pl.kernel,
    out_type=jax.ShapeDtypeStruct((NC, N_PAD), jnp.float32),
    mesh=_mesh,
    scratch_types=[
        pltpu.VMEM(((CPW + 1) * BATCH,), jnp.int32),
        pltpu.VMEM((BATCH,), jnp.float32),
        pltpu.VMEM((BATCH,), jnp.float32),
        pltpu.VMEM_SHARED((N_PAD,), jnp.float32),
    ],
    compiler_params=pltpu.CompilerParams(use_tc_tiling_on_sc=False),
)
def _deg_kernel(adj_hbm, degp_hbm, dst_v, ones_v, zero_v, deg_sh):
    c = lax.axis_index("c")
    s = lax.axis_index("s")
    wid = c * NS + s
    off, nch = _chunk_range(wid)

    one = jnp.ones((16,), jnp.float32)
    z = jnp.zeros((16,), jnp.float32)
    for j in range(BATCH // 16):
        ones_v[pl.ds(j * 16, 16)] = one
        zero_v[pl.ds(j * 16, 16)] = z

    base = s * RPW
    for k in range(RPW // BATCH):
        pltpu.sync_copy(zero_v, deg_sh.at[pl.ds(base + k * BATCH, BATCH)])

    eoff = off * BATCH

    @pl.when(wid < XTRA)
    def _():
        pltpu.sync_copy(adj_hbm.at[1, pl.ds(eoff, (CPW + 1) * BATCH)], dst_v)

    @pl.when(wid >= XTRA)
    def _():
        pltpu.sync_copy(adj_hbm.at[1, pl.ds(eoff, CPW * BATCH)],
                        dst_v.at[pl.ds(0, CPW * BATCH)])

    plsc.subcore_barrier()

    @pl.loop(0, nch)
    def _(j):
        pltpu.sync_copy(ones_v, deg_sh.at[dst_v.at[pl.ds(j * BATCH, BATCH)]], add=True)

    plsc.subcore_barrier()
    pltpu.sync_copy(deg_sh.at[pl.ds(base, RPW)],
                    degp_hbm.at[c, pl.ds(base, RPW)])


NBUF = 2


@functools.partial(
    pl.kernel,
    out_type=jax.ShapeDtypeStruct((NC, N_PAD, 128), jnp.float32),
    mesh=_mesh,
    scratch_types=[
        pltpu.VMEM(((CPW + 1) * BATCH,), jnp.int32),
        pltpu.VMEM(((CPW + 1) * BATCH,), jnp.int32),
        [pltpu.VMEM((BATCH, C2), jnp.float32)] * NBUF,
        pltpu.VMEM((BATCH, C2), jnp.float32),
        [pltpu.VMEM_SHARED((N_PAD, C2), jnp.float32)] * 2,
        pltpu.VMEM_SHARED((N_PAD, C2), jnp.float32),
        [pltpu.SemaphoreType.DMA] * NBUF,
        [pltpu.SemaphoreType.DMA] * NBUF,
        pltpu.SemaphoreType.DMA,
    ],
    compiler_params=pltpu.CompilerParams(use_tc_tiling_on_sc=False),
)
def _agg_kernel(h2s_hbm, adj_hbm, accp_hbm,
                src_v, dst_v, bufs, zbuf, h2_shs, acc_sh, gsems, ssems, psem):
    c = lax.axis_index("c")
    s = lax.axis_index("s")
    wid = c * NS + s
    off, nch = _chunk_range(wid)
    base = s * RPW

    z = jnp.zeros((16,), jnp.float32)

    @pl.loop(0, BATCH)
    def _(i):
        for j in range(C2 // 16):
            zbuf[i, pl.ds(j * 16, 16)] = z

    eoff = off * BATCH

    @pl.when(wid < XTRA)
    def _():
        pltpu.sync_copy(adj_hbm.at[0, pl.ds(eoff, (CPW + 1) * BATCH)], src_v)
        pltpu.sync_copy(adj_hbm.at[1, pl.ds(eoff, (CPW + 1) * BATCH)], dst_v)

    @pl.when(wid >= XTRA)
    def _():
        pltpu.sync_copy(adj_hbm.at[0, pl.ds(eoff, CPW * BATCH)],
                        src_v.at[pl.ds(0, CPW * BATCH)])
        pltpu.sync_copy(adj_hbm.at[1, pl.ds(eoff, CPW * BATCH)],
                        dst_v.at[pl.ds(0, CPW * BATCH)])

    pltpu.sync_copy(h2s_hbm.at[pl.ds(base, RPW), pl.ds(0, C2)],
                    h2_shs[0].at[pl.ds(base, RPW)])
    pltpu.async_copy(h2s_hbm.at[pl.ds(base, RPW), pl.ds(C2, C2)],
                     h2_shs[1].at[pl.ds(base, RPW)], psem)

    for p in range(2):
        h2_sh = h2_shs[p]
        if p == 1:
            pltpu.make_async_copy(h2s_hbm.at[pl.ds(base, RPW), pl.ds(C2, C2)],
                                  h2_shs[1].at[pl.ds(base, RPW)], psem).wait()
        for k in range(RPW // BATCH):
            pltpu.sync_copy(zbuf, acc_sh.at[pl.ds(base + k * BATCH, BATCH)])
        plsc.subcore_barrier()

        for b in range(NBUF):
            pltpu.async_copy(h2_sh.at[src_v.at[pl.ds(b * BATCH, BATCH)]], bufs[b], gsems[b])

        @pl.loop(0, CPW, step=NBUF)
        def _(j):
            for b in range(NBUF):
                jj = j + b
                pltpu.make_async_copy(h2_sh.at[src_v.at[pl.ds(jj * BATCH, BATCH)]], bufs[b], gsems[b]).wait()
                pltpu.async_copy(bufs[b], acc_sh.at[dst_v.at[pl.ds(jj * BATCH, BATCH)]], ssems[b], add=True)
            for b in range(NBUF):
                nxt = j + b + NBUF

                @pl.when(nxt < CPW)
                def _():
                    pltpu.make_async_copy(
                        bufs[b], acc_sh.at[dst_v.at[pl.ds((nxt - NBUF) * BATCH, BATCH)]], ssems[b]).wait()
                    pltpu.async_copy(h2_sh.at[src_v.at[pl.ds(nxt * BATCH, BATCH)]], bufs[b], gsems[b])

        for b in range(NBUF):
            pltpu.make_async_copy(
                bufs[b], acc_sh.at[dst_v.at[pl.ds((CPW - NBUF + b) * BATCH, BATCH)]], ssems[b]).wait()

        @pl.when(nch > CPW)
        def _():
            pltpu.sync_copy(h2_sh.at[src_v.at[pl.ds(CPW * BATCH, BATCH)]], zbuf)
            pltpu.sync_copy(zbuf, acc_sh.at[dst_v.at[pl.ds(CPW * BATCH, BATCH)]], add=True)

        plsc.subcore_barrier()
        pltpu.sync_copy(acc_sh.at[pl.ds(base, RPW)],
                        accp_hbm.at[c, pl.ds(base, RPW), pl.ds(p * C2, C2)])

        if p == 0:
            @pl.when(nch > CPW)
            def _():
                @pl.loop(0, BATCH)
                def _(i):
                    for j in range(C2 // 16):
                        zbuf[i, pl.ds(j * 16, 16)] = z


RB = 1024


def _mm_body(x_ref, w_ref, h_ref):
    h = jnp.dot(x_ref[...], w_ref[...], preferred_element_type=jnp.float32)
    h_ref[...] = jnp.concatenate([h, h], axis=1)


def _scale_body(deg_ref, h_ref, h2f_ref):
    deg = deg_ref[0] + deg_ref[1] + 1.0
    dinv = lax.rsqrt(deg)
    h2f_ref[...] = h_ref[...] * dinv[:, None]


def _fin_body(deg_ref, acc_ref, h2f_ref, b_ref, out_ref):
    deg = deg_ref[0] + deg_ref[1] + 1.0
    dinv = lax.rsqrt(deg)
    tot = acc_ref[0, :, :C] + acc_ref[1, :, :C] + h2f_ref[:, :C]
    out_ref[...] = tot * dinv[:, None] + b_ref[...]


def _tc_mm(x, W):
    return pl.pallas_call(
        _mm_body,
        grid=(N_PAD // RB,),
        in_specs=[
            pl.BlockSpec((RB, F), lambda i: (i, 0)),
            pl.BlockSpec((F, C), lambda i: (0, 0)),
        ],
        out_specs=pl.BlockSpec((RB, 128), lambda i: (i, 0)),
        out_shape=jax.ShapeDtypeStruct((N_PAD, 128), jnp.float32),
    )(x, W)


def _tc_scale(degp, h):
    return pl.pallas_call(
        _scale_body,
        grid=(N_PAD // RB,),
        in_specs=[
            pl.BlockSpec((NC, RB), lambda i: (0, i)),
            pl.BlockSpec((RB, 128), lambda i: (i, 0)),
        ],
        out_specs=pl.BlockSpec((RB, 128), lambda i: (i, 0)),
        out_shape=jax.ShapeDtypeStruct((N_PAD, 128), jnp.float32),
    )(degp, h)


def _tc_final(degp, accp, h2s, b):
    return pl.pallas_call(
        _fin_body,
        grid=(N_PAD // RB,),
        in_specs=[
            pl.BlockSpec((NC, RB), lambda i: (0, i)),
            pl.BlockSpec((NC, RB, 128), lambda i: (0, i, 0)),
            pl.BlockSpec((RB, 128), lambda i: (i, 0)),
            pl.BlockSpec((1, C), lambda i: (0, 0)),
        ],
        out_specs=pl.BlockSpec((RB, C), lambda i: (i, 0)),
        out_shape=jax.ShapeDtypeStruct((N, C), jnp.float32),
    )(degp, accp, h2s, b)


def kernel(x, adj, W, b):
    adj2d = adj.astype(jnp.int32)

    degp = _deg_kernel(adj2d)
    h = _tc_mm(x, W)
    h2s = _tc_scale(degp, h)
    accp = _agg_kernel(h2s, adj2d)
    return _tc_final(degp, accp, h2s, b.reshape(1, C))
